# Initial kernel scaffold; baseline (speedup 1.0000x reference)
import jax
import jax.numpy as jnp
from jax.experimental import pallas as pl


def kernel(X, table, W, b):
    raise NotImplementedError("write your pallas kernel here")


# speedup vs baseline: 10.9674x; 10.9674x over previous
import functools

import jax
import jax.numpy as jnp
from jax import lax
from jax.experimental import pallas as pl
from jax.experimental.pallas import tpu as pltpu
from jax.experimental.pallas import tpu_sc as plsc

EMB = 32
OUT = 128
BATCH = 16384
SEQ = 200

NC = 2
NS = 16
NW = NC * NS
ROWS_W = BATCH // NW
IDX_W = ROWS_W * SEQ
SUB = 128
CHUNK = 1024
NSUB = CHUNK // SUB
NCH = IDX_W // CHUNK


def _sc_pool_sums(x4, seg4, zeros, table):
    mesh = plsc.VectorSubcoreMesh(core_axis_name="c", subcore_axis_name="s")

    @functools.partial(
        pl.kernel,
        out_type=jax.ShapeDtypeStruct((BATCH, EMB), jnp.float32),
        mesh=mesh,
        scratch_types=[
            pltpu.VMEM((NSUB, SUB), jnp.int32),
            pltpu.VMEM((NSUB, SUB), jnp.int32),
            pltpu.VMEM((CHUNK, EMB), jnp.float32),
            pltpu.VMEM_SHARED((NS * ROWS_W, EMB), jnp.float32),
            pltpu.SemaphoreType.DMA,
        ],
        compiler_params=pltpu.CompilerParams(use_tc_tiling_on_sc=False),
    )
    def k(x_hbm, seg_hbm, z_hbm, table_hbm, out_hbm,
          idx_v, seg_v, rows_v, acc_sh, sem):
        c = lax.axis_index("c")
        s = lax.axis_index("s")
        wid = c * NS + s

        pltpu.sync_copy(z_hbm, acc_sh.at[pl.ds(s * ROWS_W, ROWS_W)])

        @pl.loop(0, NCH)
        def _(kk):
            pltpu.sync_copy(x_hbm.at[wid].at[kk], idx_v)
            pltpu.sync_copy(seg_hbm.at[s].at[kk], seg_v)
            cps = [
                pltpu.async_copy(
                    table_hbm.at[idx_v.at[j]],
                    rows_v.at[pl.ds(j * SUB, SUB)],
                    sem,
                )
                for j in range(NSUB)
            ]
            for cp in cps:
                cp.wait()
            for j in range(NSUB):
                pltpu.sync_copy(
                    rows_v.at[pl.ds(j * SUB, SUB)],
                    acc_sh.at[seg_v.at[j]],
                    add=True,
                )

        pltpu.sync_copy(
            acc_sh.at[pl.ds(s * ROWS_W, ROWS_W)],
            out_hbm.at[pl.ds(wid * ROWS_W, ROWS_W)],
        )

    return k(x4, seg4, zeros, table)


def _tc_head(hsum, w, b2):
    blk = 2048

    def body(h_ref, w_ref, b_ref, o_ref):
        h = h_ref[...] * (1.0 / SEQ)
        logits = lax.dot_general(
            h, w_ref[...], (((1,), (1,)), ((), ())),
            preferred_element_type=jnp.float32,
            precision=lax.Precision.HIGHEST,
        )
        logits = logits + b_ref[...]
        m = jnp.max(logits, axis=-1, keepdims=True)
        e = jnp.exp(logits - m)
        ls = jnp.log(jnp.sum(e, axis=-1, keepdims=True)) + m
        o_ref[...] = logits - ls

    return pl.pallas_call(
        body,
        out_shape=jax.ShapeDtypeStruct((BATCH, OUT), jnp.float32),
        grid=(BATCH // blk,),
        in_specs=[
            pl.BlockSpec((blk, EMB), lambda i: (i, 0)),
            pl.BlockSpec((OUT, EMB), lambda i: (0, 0)),
            pl.BlockSpec((1, OUT), lambda i: (0, 0)),
        ],
        out_specs=pl.BlockSpec((blk, OUT), lambda i: (i, 0)),
    )(hsum, w, b2)


def kernel(X, table, W, b):
    x4 = X.astype(jnp.int32).reshape(NW, NCH, NSUB, SUB)
    seg4 = (jnp.arange(NS * IDX_W, dtype=jnp.int32) // SEQ).reshape(
        NS, NCH, NSUB, SUB)
    zeros = jnp.zeros((ROWS_W, EMB), jnp.float32)
    hsum = _sc_pool_sums(x4, seg4, zeros, table)
    return _tc_head(hsum, W, b.reshape(1, OUT))


# speedup vs baseline: 11.1966x; 1.0209x over previous
import functools

import jax
import jax.numpy as jnp
from jax import lax
from jax.experimental import pallas as pl
from jax.experimental.pallas import tpu as pltpu
from jax.experimental.pallas import tpu_sc as plsc

EMB = 32
OUT = 128
BATCH = 16384
SEQ = 200

NC = 2
NS = 16
NW = NC * NS
ROWS_W = BATCH // NW
IDX_W = ROWS_W * SEQ
SUB = 128
CHUNK = 1024
NSUB = CHUNK // SUB
NCH = IDX_W // CHUNK


def _sc_pool_sums(x4, seg4, zeros, table):
    mesh = plsc.VectorSubcoreMesh(core_axis_name="c", subcore_axis_name="s")

    @functools.partial(
        pl.kernel,
        out_type=jax.ShapeDtypeStruct((BATCH, EMB), jnp.float32),
        mesh=mesh,
        scratch_types=[
            pltpu.VMEM((CHUNK,), jnp.int32),
            pltpu.VMEM((CHUNK,), jnp.int32),
            pltpu.VMEM((CHUNK, EMB), jnp.float32),
            pltpu.VMEM_SHARED((NS * ROWS_W, EMB), jnp.float32),
            pltpu.SemaphoreType.DMA,
        ],
        compiler_params=pltpu.CompilerParams(use_tc_tiling_on_sc=False),
    )
    def k(x_hbm, seg_hbm, z_hbm, table_hbm, out_hbm,
          idx_v, seg_v, rows_v, acc_sh, sem):
        c = lax.axis_index("c")
        s = lax.axis_index("s")
        wid = c * NS + s

        pltpu.sync_copy(z_hbm, acc_sh.at[pl.ds(s * ROWS_W, ROWS_W)])

        @pl.loop(0, NCH)
        def _(kk):
            pltpu.sync_copy(x_hbm.at[wid].at[kk], idx_v)
            pltpu.sync_copy(seg_hbm.at[s].at[kk], seg_v)
            pltpu.async_copy(table_hbm.at[idx_v], rows_v, sem).wait()
            pltpu.sync_copy(rows_v, acc_sh.at[seg_v], add=True)

        pltpu.sync_copy(
            acc_sh.at[pl.ds(s * ROWS_W, ROWS_W)],
            out_hbm.at[pl.ds(wid * ROWS_W, ROWS_W)],
        )

    return k(x4, seg4, zeros, table)


def _tc_head(hsum, w, b2):
    blk = 2048

    def body(h_ref, w_ref, b_ref, o_ref):
        h = h_ref[...] * (1.0 / SEQ)
        logits = lax.dot_general(
            h, w_ref[...], (((1,), (1,)), ((), ())),
            preferred_element_type=jnp.float32,
            precision=lax.Precision.HIGHEST,
        )
        logits = logits + b_ref[...]
        m = jnp.max(logits, axis=-1, keepdims=True)
        e = jnp.exp(logits - m)
        ls = jnp.log(jnp.sum(e, axis=-1, keepdims=True)) + m
        o_ref[...] = logits - ls

    return pl.pallas_call(
        body,
        out_shape=jax.ShapeDtypeStruct((BATCH, OUT), jnp.float32),
        grid=(BATCH // blk,),
        in_specs=[
            pl.BlockSpec((blk, EMB), lambda i: (i, 0)),
            pl.BlockSpec((OUT, EMB), lambda i: (0, 0)),
            pl.BlockSpec((1, OUT), lambda i: (0, 0)),
        ],
        out_specs=pl.BlockSpec((blk, OUT), lambda i: (i, 0)),
    )(hsum, w, b2)


def kernel(X, table, W, b):
    x4 = X.astype(jnp.int32).reshape(NW, NCH, CHUNK)
    seg4 = (jnp.arange(NS * IDX_W, dtype=jnp.int32) // SEQ).reshape(
        NS, NCH, CHUNK)
    zeros = jnp.zeros((ROWS_W, EMB), jnp.float32)
    hsum = _sc_pool_sums(x4, seg4, zeros, table)
    return _tc_head(hsum, W, b.reshape(1, OUT))


# speedup vs baseline: 11.7025x; 1.0452x over previous
import functools

import jax
import jax.numpy as jnp
from jax import lax
from jax.experimental import pallas as pl
from jax.experimental.pallas import tpu as pltpu
from jax.experimental.pallas import tpu_sc as plsc

EMB = 32
OUT = 128
BATCH = 16384
SEQ = 200

NC = 2
NS = 16
NW = NC * NS
ROWS_W = BATCH // NW
IDX_W = ROWS_W * SEQ
SUB = 128
CHUNK = 2048
NSUB = CHUNK // SUB
NCH = IDX_W // CHUNK


def _sc_pool_sums(x4, seg4, zeros, table):
    mesh = plsc.VectorSubcoreMesh(core_axis_name="c", subcore_axis_name="s")

    @functools.partial(
        pl.kernel,
        out_type=jax.ShapeDtypeStruct((BATCH, EMB), jnp.float32),
        mesh=mesh,
        scratch_types=[
            pltpu.VMEM((CHUNK,), jnp.int32),
            pltpu.VMEM((CHUNK,), jnp.int32),
            pltpu.VMEM((CHUNK, EMB), jnp.float32),
            pltpu.VMEM_SHARED((NS * ROWS_W, EMB), jnp.float32),
            pltpu.SemaphoreType.DMA,
        ],
        compiler_params=pltpu.CompilerParams(use_tc_tiling_on_sc=False),
    )
    def k(x_hbm, seg_hbm, z_hbm, table_hbm, out_hbm,
          idx_v, seg_v, rows_v, acc_sh, sem):
        c = lax.axis_index("c")
        s = lax.axis_index("s")
        wid = c * NS + s

        pltpu.sync_copy(z_hbm, acc_sh.at[pl.ds(s * ROWS_W, ROWS_W)])

        @pl.loop(0, NCH)
        def _(kk):
            pltpu.sync_copy(x_hbm.at[wid].at[kk], idx_v)
            pltpu.sync_copy(seg_hbm.at[s].at[kk], seg_v)
            pltpu.async_copy(table_hbm.at[idx_v], rows_v, sem).wait()
            pltpu.sync_copy(rows_v, acc_sh.at[seg_v], add=True)

        pltpu.sync_copy(
            acc_sh.at[pl.ds(s * ROWS_W, ROWS_W)],
            out_hbm.at[pl.ds(wid * ROWS_W, ROWS_W)],
        )

    return k(x4, seg4, zeros, table)


def _tc_head(hsum, w, b2):
    blk = 2048

    def body(h_ref, w_ref, b_ref, o_ref):
        h = h_ref[...] * (1.0 / SEQ)
        logits = lax.dot_general(
            h, w_ref[...], (((1,), (1,)), ((), ())),
            preferred_element_type=jnp.float32,
            precision=lax.Precision.HIGHEST,
        )
        logits = logits + b_ref[...]
        m = jnp.max(logits, axis=-1, keepdims=True)
        e = jnp.exp(logits - m)
        ls = jnp.log(jnp.sum(e, axis=-1, keepdims=True)) + m
        o_ref[...] = logits - ls

    return pl.pallas_call(
        body,
        out_shape=jax.ShapeDtypeStruct((BATCH, OUT), jnp.float32),
        grid=(BATCH // blk,),
        in_specs=[
            pl.BlockSpec((blk, EMB), lambda i: (i, 0)),
            pl.BlockSpec((OUT, EMB), lambda i: (0, 0)),
            pl.BlockSpec((1, OUT), lambda i: (0, 0)),
        ],
        out_specs=pl.BlockSpec((blk, OUT), lambda i: (i, 0)),
    )(hsum, w, b2)


def kernel(X, table, W, b):
    x4 = X.astype(jnp.int32).reshape(NW, NCH, CHUNK)
    seg4 = (jnp.arange(NS * IDX_W, dtype=jnp.int32) // SEQ).reshape(
        NS, NCH, CHUNK)
    zeros = jnp.zeros((ROWS_W, EMB), jnp.float32)
    hsum = _sc_pool_sums(x4, seg4, zeros, table)
    return _tc_head(hsum, W, b.reshape(1, OUT))


# speedup vs baseline: 12.9893x; 1.1100x over previous
import functools

import jax
import jax.numpy as jnp
from jax import lax
from jax.experimental import pallas as pl
from jax.experimental.pallas import tpu as pltpu
from jax.experimental.pallas import tpu_sc as plsc

EMB = 32
OUT = 128
BATCH = 16384
SEQ = 200

NC = 2
NS = 16
NW = NC * NS
ROWS_W = BATCH // NW
IDX_W = ROWS_W * SEQ
CHUNK = 1280
NCH = IDX_W // CHUNK


def _sc_pool_sums(x4, seg4, zeros, table):
    mesh = plsc.VectorSubcoreMesh(core_axis_name="c", subcore_axis_name="s")

    @functools.partial(
        pl.kernel,
        out_type=jax.ShapeDtypeStruct((BATCH, EMB), jnp.float32),
        mesh=mesh,
        scratch_types=[
            pltpu.VMEM((CHUNK,), jnp.int32),
            pltpu.VMEM((CHUNK,), jnp.int32),
            pltpu.VMEM((CHUNK,), jnp.int32),
            pltpu.VMEM((CHUNK,), jnp.int32),
            pltpu.VMEM((CHUNK, EMB), jnp.float32),
            pltpu.VMEM((CHUNK, EMB), jnp.float32),
            pltpu.VMEM_SHARED((NS * ROWS_W, EMB), jnp.float32),
            pltpu.SemaphoreType.DMA,
            pltpu.SemaphoreType.DMA,
            pltpu.SemaphoreType.DMA,
            pltpu.SemaphoreType.DMA,
            pltpu.SemaphoreType.DMA,
        ],
        compiler_params=pltpu.CompilerParams(use_tc_tiling_on_sc=False),
    )
    def k(x_hbm, seg_hbm, z_hbm, table_hbm, out_hbm,
          idx_v0, idx_v1, seg_v0, seg_v1, rows_v0, rows_v1, acc_sh,
          isem0, isem1, gsem0, gsem1, ssem):
        c = lax.axis_index("c")
        s = lax.axis_index("s")
        wid = c * NS + s

        pltpu.sync_copy(z_hbm, acc_sh.at[pl.ds(s * ROWS_W, ROWS_W)])

        ia = pltpu.async_copy(x_hbm.at[wid].at[0], idx_v0, isem0)
        ib = pltpu.async_copy(seg_hbm.at[s].at[0], seg_v0, isem0)
        ia.wait()
        ib.wait()
        pltpu.async_copy(table_hbm.at[idx_v0], rows_v0, gsem0)
        pltpu.async_copy(x_hbm.at[wid].at[1], idx_v1, isem1)
        pltpu.async_copy(seg_hbm.at[s].at[1], seg_v1, isem1)

        @pl.loop(0, NCH, step=2)
        def _(t):
            pltpu.make_async_copy(table_hbm.at[idx_v0], rows_v0, gsem0).wait()
            pltpu.make_async_copy(x_hbm.at[wid].at[t + 1], idx_v1, isem1).wait()
            pltpu.make_async_copy(seg_hbm.at[s].at[t + 1], seg_v1, isem1).wait()
            g_b = pltpu.async_copy(table_hbm.at[idx_v1], rows_v1, gsem1)
            s_a = pltpu.async_copy(rows_v0, acc_sh.at[seg_v0], ssem, add=True)
            s_a.wait()

            @pl.when(t + 2 < NCH)
            def _():
                pltpu.async_copy(x_hbm.at[wid].at[t + 2], idx_v0, isem0)
                pltpu.async_copy(seg_hbm.at[s].at[t + 2], seg_v0, isem0)

            g_b.wait()

            @pl.when(t + 2 < NCH)
            def _():
                pltpu.make_async_copy(
                    x_hbm.at[wid].at[t + 2], idx_v0, isem0).wait()
                pltpu.make_async_copy(
                    seg_hbm.at[s].at[t + 2], seg_v0, isem0).wait()
                pltpu.async_copy(table_hbm.at[idx_v0], rows_v0, gsem0)

            s_b = pltpu.async_copy(rows_v1, acc_sh.at[seg_v1], ssem, add=True)
            s_b.wait()

            @pl.when(t + 3 < NCH)
            def _():
                pltpu.async_copy(x_hbm.at[wid].at[t + 3], idx_v1, isem1)
                pltpu.async_copy(seg_hbm.at[s].at[t + 3], seg_v1, isem1)

        pltpu.sync_copy(
            acc_sh.at[pl.ds(s * ROWS_W, ROWS_W)],
            out_hbm.at[pl.ds(wid * ROWS_W, ROWS_W)],
        )

    return k(x4, seg4, zeros, table)


def _tc_head(hsum, w, b2):
    blk = 2048

    def body(h_ref, w_ref, b_ref, o_ref):
        h = h_ref[...] * (1.0 / SEQ)
        logits = lax.dot_general(
            h, w_ref[...], (((1,), (1,)), ((), ())),
            preferred_element_type=jnp.float32,
            precision=lax.Precision.HIGHEST,
        )
        logits = logits + b_ref[...]
        m = jnp.max(logits, axis=-1, keepdims=True)
        e = jnp.exp(logits - m)
        ls = jnp.log(jnp.sum(e, axis=-1, keepdims=True)) + m
        o_ref[...] = logits - ls

    return pl.pallas_call(
        body,
        out_shape=jax.ShapeDtypeStruct((BATCH, OUT), jnp.float32),
        grid=(BATCH // blk,),
        in_specs=[
            pl.BlockSpec((blk, EMB), lambda i: (i, 0)),
            pl.BlockSpec((OUT, EMB), lambda i: (0, 0)),
            pl.BlockSpec((1, OUT), lambda i: (0, 0)),
        ],
        out_specs=pl.BlockSpec((blk, OUT), lambda i: (i, 0)),
    )(hsum, w, b2)


def kernel(X, table, W, b):
    x4 = X.astype(jnp.int32).reshape(NW, NCH, CHUNK)
    seg4 = (jnp.arange(NS * IDX_W, dtype=jnp.int32) // SEQ).reshape(
        NS, NCH, CHUNK)
    zeros = jnp.zeros((ROWS_W, EMB), jnp.float32)
    hsum = _sc_pool_sums(x4, seg4, zeros, table)
    return _tc_head(hsum, W, b.reshape(1, OUT))
